# Initial kernel scaffold; baseline (speedup 1.0000x reference)
import jax
import jax.numpy as jnp
from jax.experimental import pallas as pl


def kernel(points_xyz, center_xyz, center_feature, features, affine_alpha, affine_beta):
    raise NotImplementedError("write your pallas kernel here")


# speedup vs baseline: 9.8987x; 9.8987x over previous
import jax
import jax.numpy as jnp
from jax import lax
from jax.experimental import pallas as pl
from jax.experimental.pallas import tpu as pltpu, tpu_sc as plsc

N = 32768
M = 512
KK = 512
C = 64
CF = C + 3
D = 128
CAND = 576
CANDB = 592
RB = 64
NC, NS = 2, 16
NW = NC * NS
RPW = M // NW


def _thr_body(dist_ref, thr_ref):
    dist = dist_ref[...]
    bits = lax.bitcast_convert_type(dist, jnp.int32)
    sgn = lax.shift_right_arithmetic(bits, 31)
    key = lax.bitwise_xor(bits, lax.bitwise_and(sgn, 0x7FFFFFFF))

    def body(_, lh):
        lo, hi = lh
        mid = ((lo >> 1) + (hi >> 1)) + (lo & hi & 1)
        cnt = jnp.sum((key <= mid).astype(jnp.int32), axis=1, keepdims=True)
        ge = cnt >= KK
        return jnp.where(ge, lo, mid + 1), jnp.where(ge, mid, hi)

    lo0 = jnp.full((RB, 1), jnp.iinfo(jnp.int32).min, jnp.int32)
    hi0 = jnp.full((RB, 1), jnp.iinfo(jnp.int32).max, jnp.int32)
    _, hi = lax.fori_loop(0, 32, body, (lo0, hi0))
    sgn2 = lax.shift_right_arithmetic(hi, 31)
    tbits = lax.bitwise_xor(hi, lax.bitwise_and(sgn2, 0x7FFFFFFF))
    thr = lax.bitcast_convert_type(tbits, jnp.float32)
    thr_ref[...] = jnp.broadcast_to(thr, (RB, 16))


def _thr(dist):
    return pl.pallas_call(
        _thr_body,
        grid=(M // RB,),
        in_specs=[
            pl.BlockSpec((RB, N), lambda i: (i, 0)),
        ],
        out_specs=pl.BlockSpec((RB, 16), lambda i: (i, 0)),
        out_shape=jax.ShapeDtypeStruct((M, 16), jnp.float32),
    )(dist)


def _compact_body(dist_hbm, thr_hbm, cd_hbm, cj_hbm, dbuf, cdb, cjb, thr_v,
                  off_v):
    wid = lax.axis_index("s") * NC + lax.axis_index("c")
    pltpu.sync_copy(thr_hbm.at[pl.ds(wid * (RPW * 16), RPW * 16)], thr_v)
    inf16 = jnp.full((16,), jnp.inf, jnp.float32)
    big16 = jnp.full((16,), 0x7FFFFFF0, jnp.int32)
    for rloc in range(RPW):
        row = wid * RPW + rloc
        pltpu.sync_copy(dist_hbm.at[row], dbuf)
        thrs = thr_v[pl.ds(rloc * 16, 16)]
        for q in range(CANDB // 16):
            cdb[pl.ds(q * 16, 16)] = inf16
            cjb[pl.ds(q * 16, 16)] = big16
        off_v[...] = jnp.zeros((16,), jnp.int32)

        def chunk(i, _):
            base = i * 128
            off = off_v[...]
            for u in range(8):
                b = base + u * 16
                v = dbuf[pl.ds(b, 16)]
                m = v <= thrs
                mi = m.astype(jnp.int32)
                cs = plsc.cumsum(mi)
                pos = jnp.minimum(off + cs - mi, CANDB - 1)
                idxv = lax.iota(jnp.int32, 16) + b
                plsc.store_scatter(cdb, [pos], v, mask=m)
                plsc.store_scatter(cjb, [pos], idxv, mask=m)
                off = off + plsc.all_reduce_population_count(m)
            off_v[...] = off
            return 0

        lax.fori_loop(0, N // 128, chunk, 0)
        pltpu.sync_copy(cdb.at[pl.ds(0, CAND)],
                        cd_hbm.at[pl.ds(row * CAND, CAND)])
        pltpu.sync_copy(cjb.at[pl.ds(0, CAND)],
                        cj_hbm.at[pl.ds(row * CAND, CAND)])


def _compact(dist, thr1):
    mesh = plsc.VectorSubcoreMesh(core_axis_name="c", subcore_axis_name="s",
                                  num_cores=NC, num_subcores=NS)
    fn = pl.kernel(
        _compact_body,
        out_type=[
            jax.ShapeDtypeStruct((M * CAND,), jnp.float32),
            jax.ShapeDtypeStruct((M * CAND,), jnp.int32),
        ],
        mesh=mesh,
        scratch_types=[
            pltpu.VMEM((N,), jnp.float32),
            pltpu.VMEM((CANDB,), jnp.float32),
            pltpu.VMEM((CANDB,), jnp.int32),
            pltpu.VMEM((RPW * 16,), jnp.float32),
            pltpu.VMEM((16,), jnp.int32),
        ],
        compiler_params=pltpu.CompilerParams(needs_layout_passes=False),
    )
    return fn(dist, thr1)


def _rank_body(cd_ref, cj_ref, rank_ref):
    d = cd_ref[...]
    j = cj_ref[...]
    dy = d[:, None, :]
    jy = j[:, None, :]
    TJ = 16
    for jb in range(0, CAND, TJ):
        dx = d[:, jb:jb + TJ][:, :, None]
        jx = j[:, jb:jb + TJ][:, :, None]
        pre = (dy < dx) | ((dy == dx) & (jy < jx))
        rank_ref[:, jb:jb + TJ] = jnp.sum(pre.astype(jnp.int32), axis=2)


def _rank(cd, cj):
    return pl.pallas_call(
        _rank_body,
        grid=(M // RB,),
        in_specs=[
            pl.BlockSpec((RB, CAND), lambda i: (i, 0)),
            pl.BlockSpec((RB, CAND), lambda i: (i, 0)),
        ],
        out_specs=pl.BlockSpec((RB, CAND), lambda i: (i, 0)),
        out_shape=jax.ShapeDtypeStruct((M, CAND), jnp.int32),
    )(cd, cj)


def _gather_body(rank_hbm, cj_hbm, featT_hbm, grouped_hbm,
                 rkb, cjb, idx2, rows, sem):
    wid = lax.axis_index("s") * NC + lax.axis_index("c")
    for rloc in range(RPW):
        row = wid * RPW + rloc
        pltpu.sync_copy(rank_hbm.at[pl.ds(row * CAND, CAND)], rkb)
        pltpu.sync_copy(cj_hbm.at[pl.ds(row * CAND, CAND)], cjb)
        for q in range(CAND // 16):
            r16 = rkb[pl.ds(q * 16, 16)]
            j16 = cjb[pl.ds(q * 16, 16)]
            m = r16 < KK
            rc = jnp.minimum(r16, KK - 1)
            i0 = lax.shift_right_logical(rc, 7)
            i1 = jnp.bitwise_and(rc, 127)
            plsc.store_scatter(idx2, [i0, i1], j16, mask=m)
        handles = [
            pltpu.async_copy(featT_hbm.at[idx2.at[c]],
                             rows.at[pl.ds(c * 128, 128)], sem)
            for c in range(KK // 128)
        ]
        for h in handles:
            h.wait()
        pltpu.sync_copy(rows, grouped_hbm.at[pl.ds(row * KK, KK)])


def _gather(rank, cj, featT):
    mesh = plsc.VectorSubcoreMesh(core_axis_name="c", subcore_axis_name="s",
                                  num_cores=NC, num_subcores=NS)
    fn = pl.kernel(
        _gather_body,
        out_type=jax.ShapeDtypeStruct((M * KK, D), jnp.float32),
        mesh=mesh,
        scratch_types=[
            pltpu.VMEM((CAND,), jnp.int32),
            pltpu.VMEM((CAND,), jnp.int32),
            pltpu.VMEM((KK // 128, 128), jnp.int32),
            pltpu.VMEM((KK, D), jnp.float32),
            pltpu.SemaphoreType.DMA,
        ],
        compiler_params=pltpu.CompilerParams(needs_layout_passes=False),
    )
    return fn(rank, cj, featT)


GB = 4096
GR = GB // KK


def _stat_body(g_ref, mean_ref, part_ref):
    g = g_ref[...]
    mb = mean_ref[...]
    diff = g.reshape(GR, KK, D) - mb[None, :, :]
    s = jnp.sum(diff)
    s2 = jnp.sum(diff * diff)
    part_ref[...] = jnp.concatenate(
        [s.reshape(1, 1, 1), s2.reshape(1, 1, 1)], axis=2)


def _stats(grouped, mean):
    steps = (M * KK) // GB
    return pl.pallas_call(
        _stat_body,
        grid=(steps,),
        in_specs=[
            pl.BlockSpec((GB, D), lambda i: (i, 0)),
            pl.BlockSpec((KK, D), lambda i: (0, 0)),
        ],
        out_specs=pl.BlockSpec((1, 1, 2), lambda i: (i, 0, 0)),
        out_shape=jax.ShapeDtypeStruct((steps, 1, 2), jnp.float32),
    )(grouped, mean)


def _final_body(g_ref, mean_ref, part_ref, ab_ref, out_ref):
    parts = part_ref[...]
    tot = jnp.sum(parts[:, :, 0])
    tot2 = jnp.sum(parts[:, :, 1])
    nelem = float(M * KK * CF)
    var = (tot2 - tot * tot / nelem) / (nelem - 1.0)
    denom = jnp.sqrt(var) + 1e-5
    g = g_ref[...]
    mb = mean_ref[...]
    ab = ab_ref[...]
    alpha = ab[0:1, :]
    beta = ab[1:2, :]
    diff = (g.reshape(GR, KK, D) - mb[None, :, :]).reshape(GB, D)
    y = diff * (alpha / denom) + beta
    yt = y.T
    out_ref[...] = yt[0:CF, :]


def _finalize(grouped, mean, parts, ab):
    steps = (M * KK) // GB
    return pl.pallas_call(
        _final_body,
        grid=(steps,),
        in_specs=[
            pl.BlockSpec((GB, D), lambda i: (i, 0)),
            pl.BlockSpec((KK, D), lambda i: (0, 0)),
            pl.BlockSpec((steps, 1, 2), lambda i: (0, 0, 0)),
            pl.BlockSpec((2, D), lambda i: (0, 0)),
        ],
        out_specs=pl.BlockSpec((CF, GB), lambda i: (0, i)),
        out_shape=jax.ShapeDtypeStruct((CF, M * KK), jnp.float32),
    )(grouped, mean, parts, ab)


def kernel(points_xyz, center_xyz, center_feature, features,
           affine_alpha, affine_beta):
    pts = points_xyz[0]
    cxyz = center_xyz[0]

    cc = jnp.sum(center_xyz * center_xyz, axis=-1, keepdims=True)
    pp = jnp.sum(points_xyz * points_xyz, axis=-1)[:, None, :]
    cross = jnp.einsum('bmd,bnd->bmn', center_xyz, points_xyz)
    dist = (cc + pp - 2.0 * cross)[0]

    thr = _thr(dist)
    cd, cj = _compact(dist, thr.reshape(M * 16))
    rank = _rank(cd.reshape(M, CAND), cj.reshape(M, CAND))

    featT = jnp.concatenate(
        [pts, features[0].T, jnp.zeros((N, D - CF), jnp.float32)], axis=1)
    grouped = _gather(rank.reshape(M * CAND), cj, featT)

    mean = jnp.concatenate(
        [cxyz, center_feature[0], jnp.zeros((M, D - CF), jnp.float32)],
        axis=1)
    parts = _stats(grouped, mean)
    ab = jnp.concatenate([
        jnp.pad(affine_alpha.reshape(1, CF), ((0, 0), (0, D - CF))),
        jnp.pad(affine_beta.reshape(1, CF), ((0, 0), (0, D - CF))),
    ], axis=0)
    out = _finalize(grouped, mean, parts, ab)
    return out.reshape(1, CF, M, KK)
